# baseline (speedup 1.0000x reference)
import functools

import jax
import jax.numpy as jnp
from jax import lax
from jax.experimental import pallas as pl
from jax.experimental.pallas import tpu as pltpu
from jax.experimental.pallas import tpu_sc as plsc

_B = 16384
_SPARSE_START = 13
_FIELD_NUM = 26
_VOCAB = 100000
_DIM = 16
_XCOLS = _SPARSE_START + _FIELD_NUM

_NC = 2
_NS = 16
_NW = _NC * _NS
_LANES = 16

_ROWS_PER_W = _B // _NW
_CH = 128
_NITER = _ROWS_PER_W // _CH

_OUT_COLS = _SPARSE_START + _FIELD_NUM * _DIM


def _sc_embed(x, tables):
    mesh = plsc.VectorSubcoreMesh(core_axis_name="c", subcore_axis_name="s")

    @functools.partial(
        pl.kernel,
        mesh=mesh,
        compiler_params=pltpu.CompilerParams(
            use_tc_tiling_on_sc=False, needs_layout_passes=False
        ),
        out_type=jax.ShapeDtypeStruct((_B, _OUT_COLS), jnp.float32),
        scratch_types=[
            pltpu.VMEM((_CH, _XCOLS), jnp.int32),
            pltpu.VMEM((_FIELD_NUM, _CH), jnp.int32),
            pltpu.VMEM((_FIELD_NUM, _CH, _DIM), jnp.float32),
            pltpu.VMEM((_CH, _OUT_COLS), jnp.float32),
            pltpu.SemaphoreType.DMA,
        ],
    )
    def k(x_hbm, tab_hbm, out_hbm, xs_v, idx_v, emb_v, row_v, gsem):
        wid = lax.axis_index("s") * _NC + lax.axis_index("c")
        w_base = wid * _ROWS_PER_W

        def chunk(it, carry):
            base = pl.multiple_of(w_base + it * _CH, _CH)

            pltpu.sync_copy(x_hbm.at[pl.ds(base, _CH)], xs_v)

            row_iota = lax.iota(jnp.int32, _LANES)
            for f in range(_FIELD_NUM):
                col = jnp.full((_LANES,), _SPARSE_START + f, jnp.int32)
                for j in range(_CH // _LANES):
                    rows = row_iota + (j * _LANES)
                    vals = plsc.load_gather(xs_v, [rows, col])
                    idx_v[f, pl.ds(j * _LANES, _LANES)] = vals

            handles = []
            for f in range(_FIELD_NUM):
                handles.append(
                    pltpu.async_copy(
                        tab_hbm.at[f].at[idx_v.at[f]], emb_v.at[f], gsem
                    )
                )
            for h in handles:
                h.wait()

            def put_row(r, c):
                d = xs_v[r, pl.ds(0, _LANES)].astype(jnp.float32)
                row_v[r, pl.ds(0, _LANES)] = d
                for f in range(_FIELD_NUM):
                    row_v[r, pl.ds(_SPARSE_START + f * _DIM, _DIM)] = (
                        emb_v[f, r, pl.ds(0, _DIM)]
                    )
                return c

            lax.fori_loop(0, _CH, put_row, 0)

            pltpu.sync_copy(row_v, out_hbm.at[pl.ds(base, _CH)])
            return carry

        lax.fori_loop(0, _NITER, chunk, 0)

    return k(x, tables)


def kernel(X, tables):
    return _sc_embed(X, tables)
